# Initial kernel scaffold; baseline (speedup 1.0000x reference)
import jax
import jax.numpy as jnp
from jax.experimental import pallas as pl


def kernel(nbf_score, simkgc_score):
    raise NotImplementedError("write your pallas kernel here")


# speedup vs baseline: 11.4486x; 11.4486x over previous
import functools

import jax
import jax.numpy as jnp
from jax import lax
from jax.experimental import pallas as pl
from jax.experimental.pallas import tpu as pltpu

_K = 100
_ROWS_PER_BLOCK = 8


def _f32_to_ordered_i32(x):
    s = lax.bitcast_convert_type(x, jnp.int32)
    return s ^ (lax.shift_right_arithmetic(s, 31) & jnp.int32(0x7FFFFFFF))


def _ordered_i32_to_f32(k):
    s = k ^ (lax.shift_right_arithmetic(k, 31) & jnp.int32(0x7FFFFFFF))
    return lax.bitcast_convert_type(s, jnp.float32)


def _tc_body(nbf_ref, sim_ref, out_ref):
    nbf = nbf_ref[...]
    sim = sim_ref[...]

    min_n = jnp.min(nbf, axis=1, keepdims=True)
    d_n = nbf - min_n
    den_n = jnp.max(d_n, axis=1, keepdims=True)
    min_s = jnp.min(sim, axis=1, keepdims=True)
    d_s = sim - min_s
    den_s = jnp.max(d_s, axis=1, keepdims=True)

    keys = _f32_to_ordered_i32(nbf)
    lo = jnp.min(keys, axis=1, keepdims=True)
    hi = jnp.max(keys, axis=1, keepdims=True)

    def step(_, carry):
        lo, hi = carry
        x = lo ^ hi
        mid = (lo & hi) + lax.shift_right_arithmetic(x, 1) + (x & 1)
        cnt = jnp.sum((keys >= mid).astype(jnp.int32), axis=1, keepdims=True)
        ge = cnt >= _K
        return jnp.where(ge, mid, lo), jnp.where(ge, hi, mid - 1)

    lo, hi = lax.fori_loop(0, 32, step, (lo, hi))

    nbf_n = d_n / den_n
    sim_n = d_s / den_s
    thresh = jnp.max(
        jnp.where(keys <= lo, nbf_n, -jnp.inf), axis=1, keepdims=True
    )
    out_ref[...] = nbf_n + jnp.where(
        nbf_n >= thresh, 1000.0 * (1.0 + sim_n), 0.0
    )


@jax.jit
def kernel(nbf_score, simkgc_score):
    b, n = nbf_score.shape
    grid = (b // _ROWS_PER_BLOCK,)
    spec = pl.BlockSpec((_ROWS_PER_BLOCK, n), lambda i: (i, 0))
    return pl.pallas_call(
        _tc_body,
        grid=grid,
        in_specs=[spec, spec],
        out_specs=spec,
        out_shape=jax.ShapeDtypeStruct((b, n), jnp.float32),
    )(nbf_score, simkgc_score)
